# baseline (speedup 1.0000x reference)
import jax
import jax.numpy as jnp
from jax import lax
from jax.experimental import pallas as pl
from jax.experimental.pallas import tpu as pltpu
from jax.experimental.pallas import tpu_sc as plsc

_N = 10000
_E = 320000
_D = 128
_R = 4
_BN_EPS = 1e-5

_NQ = 4
_QR = _N // _NQ
_ROWS = _R * _QR
_ROWS_PAD = 10240
_DUMMY = _ROWS
_NTILES = 16
_NW = 32
_RPT = _ROWS_PAD // _NTILES
_CH = 128
_EPW = 10240
_E_PAD = _EPW * _NW
_IGRP = 8
_NIG = _EPW // (_IGRP * _CH)
_BLK = 1024
_NCB = _BLK // _CH
_CAP = _EPW + _BLK
_TRASH = _CAP
_CAP2 = _CAP + 16
_ZROWS = 32


def _bin_kernel(src_hbm, dst_hbm, typ_hbm, bsrc_hbm, bslot_hbm, cnt_hbm,
                srcb, dstb, typb, ss0, ss1, ss2, ss3, sl0, sl1, sl2, sl3,
                cbuf):
    st_src = (ss0, ss1, ss2, ss3)
    st_slot = (sl0, sl1, sl2, sl3)
    c = lax.axis_index("c")
    s = lax.axis_index("s")
    w = c * _NTILES + s
    ebase = w * _EPW
    iota16 = lax.iota(jnp.int32, 16)

    def _igroup(g, cnts):
        goff = ebase + g * _IGRP * _CH
        pltpu.sync_copy(src_hbm.at[pl.ds(goff, _IGRP * _CH)], srcb)
        pltpu.sync_copy(dst_hbm.at[pl.ds(goff, _IGRP * _CH)], dstb)
        pltpu.sync_copy(typ_hbm.at[pl.ds(goff, _IGRP * _CH)], typb)

        def _vec(i, cnts):
            sv = srcb[pl.ds(i * 16, 16)]
            d = dstb[pl.ds(i * 16, 16)]
            tt = typb[pl.ds(i * 16, 16)]
            qv = d // _QR
            slot = tt * _QR + (d - qv * _QR)
            new = []
            for qq in range(_NQ):
                m = qv == qq
                key = jnp.where(m, iota16, 16 + iota16)
                _, csv = plsc.sort_key_val(key, sv)
                _, cslot = plsc.sort_key_val(key, slot)
                st_src[qq][pl.ds(cnts[qq], 16)] = csv
                st_slot[qq][pl.ds(cnts[qq], 16)] = cslot
                new.append(cnts[qq] + plsc.all_reduce_population_count(m)[0])
            return tuple(new)
        return lax.fori_loop(0, _IGRP * _CH // 16, _vec, cnts)

    cnts = lax.fori_loop(0, _NIG, _igroup, (0, 0, 0, 0))

    for qq in range(_NQ):
        cq = cnts[qq]
        npad = ((cq + _BLK - 1) // _BLK) * _BLK

        def _padb(t, carry):
            st_src[qq][pl.ds(cq + t * 16, 16)] = jnp.zeros((16,), jnp.int32)
            st_slot[qq][pl.ds(cq + t * 16, 16)] = (
                _DUMMY + (t % 15) * 16 + iota16)
            return carry
        lax.fori_loop(0, _BLK // 16, _padb, 0)

        nblk = npad // _BLK

        def _flush(b, carry):
            boff = b * _BLK
            pltpu.sync_copy(st_src[qq].at[pl.ds(boff, _BLK)],
                            bsrc_hbm.at[w, qq, pl.ds(boff, _BLK)])
            pltpu.sync_copy(st_slot[qq].at[pl.ds(boff, _BLK)],
                            bslot_hbm.at[w, qq, pl.ds(boff, _BLK)])
            return carry
        lax.fori_loop(0, nblk, _flush, 0)

        cbuf[qq, pl.ds(0, 16)] = jnp.broadcast_to(nblk, (16,))
    pltpu.sync_copy(cbuf, cnt_hbm.at[w])


def _run_bin(src2d, dst2d, typ2d):
    mesh = plsc.VectorSubcoreMesh(core_axis_name="c", subcore_axis_name="s",
                                  num_cores=2)
    fn = pl.kernel(
        _bin_kernel,
        mesh=mesh,
        compiler_params=pltpu.CompilerParams(needs_layout_passes=False),
        out_type=(
            jax.ShapeDtypeStruct((_NW, _NQ, _CAP), jnp.int32),
            jax.ShapeDtypeStruct((_NW, _NQ, _CAP), jnp.int32),
            jax.ShapeDtypeStruct((_NW, _NQ, 16), jnp.int32),
        ),
        scratch_types=[
            pltpu.VMEM((_IGRP * _CH,), jnp.int32),
            pltpu.VMEM((_IGRP * _CH,), jnp.int32),
            pltpu.VMEM((_IGRP * _CH,), jnp.int32),
            pltpu.VMEM((_CAP2,), jnp.int32),
            pltpu.VMEM((_CAP2,), jnp.int32),
            pltpu.VMEM((_CAP2,), jnp.int32),
            pltpu.VMEM((_CAP2,), jnp.int32),
            pltpu.VMEM((_CAP2,), jnp.int32),
            pltpu.VMEM((_CAP2,), jnp.int32),
            pltpu.VMEM((_CAP2,), jnp.int32),
            pltpu.VMEM((_CAP2,), jnp.int32),
            pltpu.VMEM((_NQ, 16), jnp.int32),
        ],
    )
    return fn(src2d, dst2d, typ2d)


def _agg_kernel(bsrc_hbm, bslot_hbm, cnt_hbm, x_hbm, out_hbm,
                srcg, slotg, rows0, rows1, zbuf, cbuf, agg_sh,
                gsem0, gsem1, ssem0, ssem1):
    c = lax.axis_index("c")
    s = lax.axis_index("s")
    rows = (rows0, rows1)
    gsem = (gsem0, gsem1)
    ssem = (ssem0, ssem1)

    def _zbody(i, carry):
        for l in range(_D // 16):
            zbuf[i, pl.ds(l * 16, 16)] = jnp.zeros((16,), jnp.float32)
        return carry
    lax.fori_loop(0, _ZROWS, _zbody, 0)

    def _fire_gather(t):
        pltpu.async_copy(x_hbm.at[srcg.at[t]], rows[t % 2], gsem[t % 2])

    def _wait_gather(t):
        pltpu.make_async_copy(x_hbm.at[srcg.at[t]], rows[t % 2],
                              gsem[t % 2]).wait()

    def _fire_scatter(t):
        pltpu.async_copy(rows[t % 2], agg_sh.at[slotg.at[t]], ssem[t % 2],
                         add=True)

    def _wait_scatter(t):
        pltpu.make_async_copy(rows[t % 2], agg_sh.at[slotg.at[t]],
                              ssem[t % 2]).wait()

    for p in range(2):
        q = c * 2 + p

        for b in range(_RPT // _ZROWS):
            zoff = pl.multiple_of(s * _RPT + b * _ZROWS, _ZROWS)
            pltpu.sync_copy(zbuf, agg_sh.at[pl.ds(zoff, _ZROWS)])
        plsc.subcore_barrier()

        for wh in range(2):
            w = 2 * s + wh
            pltpu.sync_copy(cnt_hbm.at[w], cbuf)
            nblk = cbuf[q, pl.ds(0, 16)][0]

            def _super(b, carry):
                boff = b * _NCB
                pltpu.sync_copy(bsrc_hbm.at[w, q, pl.ds(boff, _NCB)], srcg)
                pltpu.sync_copy(bslot_hbm.at[w, q, pl.ds(boff, _NCB)], slotg)
                for t in range(_NCB):
                    if t >= 2:
                        _wait_scatter(t - 2)
                    _fire_gather(t)
                    if t >= 1:
                        _wait_gather(t - 1)
                        _fire_scatter(t - 1)
                _wait_gather(_NCB - 1)
                _fire_scatter(_NCB - 1)
                _wait_scatter(_NCB - 2)
                _wait_scatter(_NCB - 1)
                return carry
            lax.fori_loop(0, nblk, _super, 0)
        plsc.subcore_barrier()

        woff = pl.multiple_of(s * _RPT, _RPT)
        pltpu.sync_copy(agg_sh.at[pl.ds(woff, _RPT)],
                        out_hbm.at[q, pl.ds(woff, _RPT)])


def _run_agg(bsrc, bslot, cnt, x):
    mesh = plsc.VectorSubcoreMesh(core_axis_name="c", subcore_axis_name="s",
                                  num_cores=2)
    fn = pl.kernel(
        _agg_kernel,
        mesh=mesh,
        out_type=jax.ShapeDtypeStruct((_NQ, _ROWS_PAD, _D), jnp.float32),
        scratch_types=[
            pltpu.VMEM((_NCB, _CH), jnp.int32),
            pltpu.VMEM((_NCB, _CH), jnp.int32),
            pltpu.VMEM((_CH, _D), jnp.float32),
            pltpu.VMEM((_CH, _D), jnp.float32),
            pltpu.VMEM((_ZROWS, _D), jnp.float32),
            pltpu.VMEM((_NQ, 16), jnp.int32),
            pltpu.VMEM_SHARED((_ROWS_PAD, _D), jnp.float32),
            pltpu.SemaphoreType.DMA,
            pltpu.SemaphoreType.DMA,
            pltpu.SemaphoreType.DMA,
            pltpu.SemaphoreType.DMA,
        ],
    )
    return fn(bsrc, bslot, cnt, x)


def _tc_body(x_ref, a_ref, wsl_ref, bsl_ref, w1_ref, b1_ref, g_ref, be_ref,
             w2_ref, b2_ref, o_ref):
    x = x_ref[...]
    acc = jnp.dot(x, wsl_ref[...],
                  preferred_element_type=jnp.float32) + bsl_ref[...][None, :]
    for r in range(_R):
        agg = jnp.concatenate(
            [a_ref[q, r * _QR:(r + 1) * _QR, :] for q in range(_NQ)], axis=0)
        h = x + agg
        h = jnp.dot(h, w1_ref[r],
                    preferred_element_type=jnp.float32) + b1_ref[r][None, :]
        mean = jnp.mean(h, axis=0)
        hc = h - mean[None, :]
        var = jnp.mean(hc * hc, axis=0)
        inv = lax.rsqrt(var + _BN_EPS)
        h = hc * (inv * g_ref[r])[None, :] + be_ref[r][None, :]
        h = jnp.maximum(h, 0.0)
        acc = acc + jnp.dot(h, w2_ref[r],
                            preferred_element_type=jnp.float32) + b2_ref[r][None, :]
    o_ref[...] = acc


def _tc_mlp(x, agg, W_sl, b_sl, W1, b1, gamma, beta, W2, b2):
    return pl.pallas_call(
        _tc_body,
        out_shape=jax.ShapeDtypeStruct((_N, _D), jnp.float32),
    )(x, agg, W_sl, b_sl, W1, b1, gamma, beta, W2, b2)


def kernel(x, edge_index, edge_type, W_sl, b_sl, W1, b1, gamma, beta, W2, b2):
    src = edge_index[0]
    dst = edge_index[1]
    pad = _E_PAD - _E
    src_p = jnp.concatenate([src, jnp.zeros((pad,), jnp.int32)])
    dst_p = jnp.concatenate([dst, jnp.full((pad,), _N, jnp.int32)])
    typ_p = jnp.concatenate([edge_type, jnp.zeros((pad,), jnp.int32)])
    src2d = src_p.reshape(_E_PAD // _CH, _CH)
    dst2d = dst_p.reshape(_E_PAD // _CH, _CH)
    typ2d = typ_p.reshape(_E_PAD // _CH, _CH)
    bsrc, bslot, cnt = _run_bin(src_p, dst_p, typ_p)
    bsrc4 = bsrc.reshape(_NW, _NQ, _CAP // _CH, _CH)
    bslot4 = bslot.reshape(_NW, _NQ, _CAP // _CH, _CH)
    agg = _run_agg(bsrc4, bslot4, cnt, x)
    return _tc_mlp(x, agg, W_sl, b_sl, W1, b1, gamma, beta, W2, b2)


# speedup vs baseline: 1.0006x; 1.0006x over previous
import jax
import jax.numpy as jnp
from jax import lax
from jax.experimental import pallas as pl
from jax.experimental.pallas import tpu as pltpu
from jax.experimental.pallas import tpu_sc as plsc

_N = 10000
_E = 320000
_D = 128
_R = 4
_BN_EPS = 1e-5

_NQ = 4
_QR = _N // _NQ
_ROWS = _R * _QR
_ROWS_PAD = 10240
_DUMMY = _ROWS
_NTILES = 16
_NW = 32
_RPT = _ROWS_PAD // _NTILES
_CH = 128
_EPW = 10240
_E_PAD = _EPW * _NW
_IGRP = 8
_NIG = _EPW // (_IGRP * _CH)
_BLK = 1024
_NCB = _BLK // _CH
_CAP = _EPW + _BLK
_TRASH = _CAP
_CAP2 = _CAP + 16
_ZROWS = 32


def _bin_kernel(src_hbm, dst_hbm, typ_hbm, bsrc_hbm, bslot_hbm, cnt_hbm,
                srcb, dstb, typb, ss0, ss1, ss2, ss3, sl0, sl1, sl2, sl3,
                cbuf):
    st_src = (ss0, ss1, ss2, ss3)
    st_slot = (sl0, sl1, sl2, sl3)
    c = lax.axis_index("c")
    s = lax.axis_index("s")
    w = c * _NTILES + s
    ebase = w * _EPW
    iota16 = lax.iota(jnp.int32, 16)

    def _igroup(g, cnts):
        goff = ebase + g * _IGRP * _CH
        pltpu.sync_copy(src_hbm.at[pl.ds(goff, _IGRP * _CH)], srcb)
        pltpu.sync_copy(dst_hbm.at[pl.ds(goff, _IGRP * _CH)], dstb)
        pltpu.sync_copy(typ_hbm.at[pl.ds(goff, _IGRP * _CH)], typb)

        def _vec(i, cnts):
            sv = srcb[pl.ds(i * 16, 16)]
            d = dstb[pl.ds(i * 16, 16)]
            tt = typb[pl.ds(i * 16, 16)]
            qv = d // _QR
            slot = tt * _QR + (d - qv * _QR)
            new = []
            for qq in range(_NQ):
                m = qv == qq
                key = jnp.where(m, iota16, 16 + iota16)
                _, csv = plsc.sort_key_val(key, sv)
                _, cslot = plsc.sort_key_val(key, slot)
                st_src[qq][pl.ds(cnts[qq], 16)] = csv
                st_slot[qq][pl.ds(cnts[qq], 16)] = cslot
                new.append(cnts[qq] + plsc.all_reduce_population_count(m)[0])
            return tuple(new)
        return lax.fori_loop(0, _IGRP * _CH // 16, _vec, cnts)

    cnts = lax.fori_loop(0, _NIG, _igroup, (0, 0, 0, 0))

    for qq in range(_NQ):
        cq = cnts[qq]
        npad = ((cq + _BLK - 1) // _BLK) * _BLK

        def _padb(t, carry):
            st_src[qq][pl.ds(cq + t * 16, 16)] = jnp.zeros((16,), jnp.int32)
            st_slot[qq][pl.ds(cq + t * 16, 16)] = (
                _DUMMY + (t % 15) * 16 + iota16)
            return carry
        lax.fori_loop(0, _BLK // 16, _padb, 0)

        nblk = npad // _BLK

        def _flush(b, carry):
            boff = b * _BLK
            pltpu.sync_copy(st_src[qq].at[pl.ds(boff, _BLK)],
                            bsrc_hbm.at[w, qq, pl.ds(boff, _BLK)])
            pltpu.sync_copy(st_slot[qq].at[pl.ds(boff, _BLK)],
                            bslot_hbm.at[w, qq, pl.ds(boff, _BLK)])
            return carry
        lax.fori_loop(0, nblk, _flush, 0)

        cbuf[qq, pl.ds(0, 16)] = jnp.broadcast_to(nblk, (16,))
    pltpu.sync_copy(cbuf, cnt_hbm.at[w])


def _run_bin(src2d, dst2d, typ2d):
    mesh = plsc.VectorSubcoreMesh(core_axis_name="c", subcore_axis_name="s",
                                  num_cores=2)
    fn = pl.kernel(
        _bin_kernel,
        mesh=mesh,
        compiler_params=pltpu.CompilerParams(needs_layout_passes=False),
        out_type=(
            jax.ShapeDtypeStruct((_NW, _NQ, _CAP), jnp.int32),
            jax.ShapeDtypeStruct((_NW, _NQ, _CAP), jnp.int32),
            jax.ShapeDtypeStruct((_NW, _NQ, 16), jnp.int32),
        ),
        scratch_types=[
            pltpu.VMEM((_IGRP * _CH,), jnp.int32),
            pltpu.VMEM((_IGRP * _CH,), jnp.int32),
            pltpu.VMEM((_IGRP * _CH,), jnp.int32),
            pltpu.VMEM((_CAP2,), jnp.int32),
            pltpu.VMEM((_CAP2,), jnp.int32),
            pltpu.VMEM((_CAP2,), jnp.int32),
            pltpu.VMEM((_CAP2,), jnp.int32),
            pltpu.VMEM((_CAP2,), jnp.int32),
            pltpu.VMEM((_CAP2,), jnp.int32),
            pltpu.VMEM((_CAP2,), jnp.int32),
            pltpu.VMEM((_CAP2,), jnp.int32),
            pltpu.VMEM((_NQ, 16), jnp.int32),
        ],
    )
    return fn(src2d, dst2d, typ2d)


def _agg_kernel(bsrc_hbm, bslot_hbm, cnt_hbm, x_hbm, out_hbm,
                srcg, slotg, rows0, rows1, zbuf, cbuf, agg_sh,
                gsem0, gsem1, ssem0, ssem1):
    c = lax.axis_index("c")
    s = lax.axis_index("s")
    rows = (rows0, rows1)
    gsem = (gsem0, gsem1)
    ssem = (ssem0, ssem1)

    def _zbody(i, carry):
        for l in range(_D // 16):
            zbuf[i, pl.ds(l * 16, 16)] = jnp.zeros((16,), jnp.float32)
        return carry
    lax.fori_loop(0, _ZROWS, _zbody, 0)

    def _fire_gather(t):
        pltpu.async_copy(x_hbm.at[srcg.at[t]], rows[t % 2], gsem[t % 2])

    def _wait_gather(t):
        pltpu.make_async_copy(x_hbm.at[srcg.at[t]], rows[t % 2],
                              gsem[t % 2]).wait()

    def _fire_scatter(t):
        pltpu.async_copy(rows[t % 2], agg_sh.at[slotg.at[t]], ssem[t % 2],
                         add=True)

    def _wait_scatter(t):
        pltpu.make_async_copy(rows[t % 2], agg_sh.at[slotg.at[t]],
                              ssem[t % 2]).wait()

    for p in range(2):
        q = c * 2 + p

        for b in range(_RPT // _ZROWS):
            zoff = pl.multiple_of(s * _RPT + b * _ZROWS, _ZROWS)
            pltpu.sync_copy(zbuf, agg_sh.at[pl.ds(zoff, _ZROWS)])
        plsc.subcore_barrier()

        for wh in range(2):
            w = 2 * s + wh
            pltpu.sync_copy(cnt_hbm.at[w], cbuf)
            nblk = cbuf[q, pl.ds(0, 16)][0]

            def _super(b, carry):
                @pl.when(b < nblk)
                def _():
                    boff = b * _NCB
                    pltpu.sync_copy(bsrc_hbm.at[w, q, pl.ds(boff, _NCB)],
                                    srcg)
                    pltpu.sync_copy(bslot_hbm.at[w, q, pl.ds(boff, _NCB)],
                                    slotg)
                    for t in range(_NCB):
                        if t >= 2:
                            _wait_scatter(t - 2)
                        _fire_gather(t)
                        if t >= 1:
                            _wait_gather(t - 1)
                            _fire_scatter(t - 1)
                    _wait_gather(_NCB - 1)
                    _fire_scatter(_NCB - 1)
                    _wait_scatter(_NCB - 2)
                    _wait_scatter(_NCB - 1)
                return carry
            lax.fori_loop(0, _CAP // _BLK, _super, 0)
        plsc.subcore_barrier()

        woff = pl.multiple_of(s * _RPT, _RPT)
        pltpu.sync_copy(agg_sh.at[pl.ds(woff, _RPT)],
                        out_hbm.at[q, pl.ds(woff, _RPT)])


def _run_agg(bsrc, bslot, cnt, x):
    mesh = plsc.VectorSubcoreMesh(core_axis_name="c", subcore_axis_name="s",
                                  num_cores=2)
    fn = pl.kernel(
        _agg_kernel,
        mesh=mesh,
        out_type=jax.ShapeDtypeStruct((_NQ, _ROWS_PAD, _D), jnp.float32),
        scratch_types=[
            pltpu.VMEM((_NCB, _CH), jnp.int32),
            pltpu.VMEM((_NCB, _CH), jnp.int32),
            pltpu.VMEM((_CH, _D), jnp.float32),
            pltpu.VMEM((_CH, _D), jnp.float32),
            pltpu.VMEM((_ZROWS, _D), jnp.float32),
            pltpu.VMEM((_NQ, 16), jnp.int32),
            pltpu.VMEM_SHARED((_ROWS_PAD, _D), jnp.float32),
            pltpu.SemaphoreType.DMA,
            pltpu.SemaphoreType.DMA,
            pltpu.SemaphoreType.DMA,
            pltpu.SemaphoreType.DMA,
        ],
    )
    return fn(bsrc, bslot, cnt, x)


def _tc_body(x_ref, a_ref, wsl_ref, bsl_ref, w1_ref, b1_ref, g_ref, be_ref,
             w2_ref, b2_ref, o_ref):
    x = x_ref[...]
    acc = jnp.dot(x, wsl_ref[...],
                  preferred_element_type=jnp.float32) + bsl_ref[...][None, :]
    for r in range(_R):
        agg = jnp.concatenate(
            [a_ref[q, r * _QR:(r + 1) * _QR, :] for q in range(_NQ)], axis=0)
        h = x + agg
        h = jnp.dot(h, w1_ref[r],
                    preferred_element_type=jnp.float32) + b1_ref[r][None, :]
        mean = jnp.mean(h, axis=0)
        hc = h - mean[None, :]
        var = jnp.mean(hc * hc, axis=0)
        inv = lax.rsqrt(var + _BN_EPS)
        h = hc * (inv * g_ref[r])[None, :] + be_ref[r][None, :]
        h = jnp.maximum(h, 0.0)
        acc = acc + jnp.dot(h, w2_ref[r],
                            preferred_element_type=jnp.float32) + b2_ref[r][None, :]
    o_ref[...] = acc


def _tc_mlp(x, agg, W_sl, b_sl, W1, b1, gamma, beta, W2, b2):
    return pl.pallas_call(
        _tc_body,
        out_shape=jax.ShapeDtypeStruct((_N, _D), jnp.float32),
    )(x, agg, W_sl, b_sl, W1, b1, gamma, beta, W2, b2)


def kernel(x, edge_index, edge_type, W_sl, b_sl, W1, b1, gamma, beta, W2, b2):
    src = edge_index[0]
    dst = edge_index[1]
    pad = _E_PAD - _E
    src_p = jnp.concatenate([src, jnp.zeros((pad,), jnp.int32)])
    dst_p = jnp.concatenate([dst, jnp.full((pad,), _N, jnp.int32)])
    typ_p = jnp.concatenate([edge_type, jnp.zeros((pad,), jnp.int32)])
    src2d = src_p.reshape(_E_PAD // _CH, _CH)
    dst2d = dst_p.reshape(_E_PAD // _CH, _CH)
    typ2d = typ_p.reshape(_E_PAD // _CH, _CH)
    bsrc, bslot, cnt = _run_bin(src_p, dst_p, typ_p)
    bsrc4 = bsrc.reshape(_NW, _NQ, _CAP // _CH, _CH)
    bslot4 = bslot.reshape(_NW, _NQ, _CAP // _CH, _CH)
    agg = _run_agg(bsrc4, bslot4, cnt, x)
    return _tc_mlp(x, agg, W_sl, b_sl, W1, b1, gamma, beta, W2, b2)


# speedup vs baseline: 1.7096x; 1.7085x over previous
import jax
import jax.numpy as jnp
from jax import lax
from jax.experimental import pallas as pl
from jax.experimental.pallas import tpu as pltpu
from jax.experimental.pallas import tpu_sc as plsc

_N = 10000
_E = 320000
_D = 128
_R = 4
_BN_EPS = 1e-5

_NQ = 4
_QR = _N // _NQ
_ROWS = _R * _QR
_ROWS_PAD = 10240
_DUMMY = _ROWS
_NTILES = 16
_RPT = _ROWS_PAD // _NTILES
_CH = 128
_EPT = 20480
_E_PAD = _EPT * _NTILES
_NCH = _EPT // _CH
_GRP = 8
_NGRP = _NCH // _GRP
_ZROWS = 32


def _sc_agg_kernel(src_hbm, dst_hbm, typ_hbm, x_hbm, zero_hbm, out_hbm,
                   srcg0, srcg1, dstg0, dstg1, typg0, typg1,
                   slotg0, slotg1, rows0, rows1, agg_sh,
                   gsem0, gsem1, ssem0, ssem1):
    c = lax.axis_index("c")
    s = lax.axis_index("s")
    row_base = s * _NCH
    srcg = (srcg0, srcg1)
    dstg = (dstg0, dstg1)
    typg = (typg0, typg1)
    slotg = (slotg0, slotg1)
    rows = (rows0, rows1)
    gsem = (gsem0, gsem1)
    ssem = (ssem0, ssem1)

    def _fire_gather(u, t):
        pltpu.async_copy(x_hbm.at[srcg[u].at[t]], rows[t % 2], gsem[t % 2])

    def _wait_gather(u, t):
        pltpu.make_async_copy(x_hbm.at[srcg[u].at[t]], rows[t % 2],
                              gsem[t % 2]).wait()

    def _fire_scatter(u, t):
        pltpu.async_copy(rows[t % 2], agg_sh.at[slotg[u].at[t]],
                         ssem[t % 2], add=True)

    def _wait_scatter(u, t):
        pltpu.make_async_copy(rows[t % 2], agg_sh.at[slotg[u].at[t]],
                              ssem[t % 2]).wait()

    for p in range(2):
        q = c * 2 + p
        base_row = q * _QR

        zoff = pl.multiple_of(s * _RPT, _RPT)
        pltpu.sync_copy(zero_hbm, agg_sh.at[pl.ds(zoff, _RPT)])
        plsc.subcore_barrier()

        iota16 = lax.iota(jnp.int32, 16)

        def _slots(u, t):
            for k in range(_CH // 16):
                d = dstg[u][t, pl.ds(k * 16, 16)]
                tt = typg[u][t, pl.ds(k * 16, 16)]
                in_r = (d >= base_row) & (d < base_row + _QR)
                dummy = _DUMMY + (k % 15) * 16 + iota16
                slot = jnp.where(in_r, tt * _QR + (d - base_row), dummy)
                slotg[u][t, pl.ds(k * 16, 16)] = slot

        def _group(u, g):
            grow = row_base + g * _GRP
            pltpu.sync_copy(src_hbm.at[pl.ds(grow, _GRP)], srcg[u])
            pltpu.sync_copy(dst_hbm.at[pl.ds(grow, _GRP)], dstg[u])
            pltpu.sync_copy(typ_hbm.at[pl.ds(grow, _GRP)], typg[u])
            for t in range(_GRP):
                if t >= 2:
                    _wait_scatter(u, t - 2)
                else:
                    @pl.when(g > 0)
                    def _():
                        _wait_scatter(u ^ 1, _GRP - 2 + t)
                _slots(u, t)
                _fire_gather(u, t)
                if t >= 1:
                    _wait_gather(u, t - 1)
                    _fire_scatter(u, t - 1)
                else:
                    @pl.when(g > 0)
                    def _():
                        _wait_gather(u ^ 1, _GRP - 1)
                        _fire_scatter(u ^ 1, _GRP - 1)

        def _gpair(j, carry):
            _group(0, 2 * j)
            _group(1, 2 * j + 1)
            return carry
        lax.fori_loop(0, _NGRP // 2, _gpair, 0)
        _wait_gather(1, _GRP - 1)
        _fire_scatter(1, _GRP - 1)
        _wait_scatter(1, _GRP - 2)
        _wait_scatter(1, _GRP - 1)
        plsc.subcore_barrier()

        woff = pl.multiple_of(s * _RPT, _RPT)
        pltpu.sync_copy(agg_sh.at[pl.ds(woff, _RPT)],
                        out_hbm.at[q, pl.ds(woff, _RPT)])


def _run_sc_agg(src2d, dst2d, typ2d, x, zeros):
    mesh = plsc.VectorSubcoreMesh(core_axis_name="c", subcore_axis_name="s",
                                  num_cores=2)
    fn = pl.kernel(
        _sc_agg_kernel,
        mesh=mesh,
        out_type=jax.ShapeDtypeStruct((_NQ, _ROWS_PAD, _D), jnp.float32),
        scratch_types=[
            pltpu.VMEM((_GRP, _CH), jnp.int32),
            pltpu.VMEM((_GRP, _CH), jnp.int32),
            pltpu.VMEM((_GRP, _CH), jnp.int32),
            pltpu.VMEM((_GRP, _CH), jnp.int32),
            pltpu.VMEM((_GRP, _CH), jnp.int32),
            pltpu.VMEM((_GRP, _CH), jnp.int32),
            pltpu.VMEM((_GRP, _CH), jnp.int32),
            pltpu.VMEM((_GRP, _CH), jnp.int32),
            pltpu.VMEM((_CH, _D), jnp.float32),
            pltpu.VMEM((_CH, _D), jnp.float32),
            pltpu.VMEM_SHARED((_ROWS_PAD, _D), jnp.float32),
            pltpu.SemaphoreType.DMA,
            pltpu.SemaphoreType.DMA,
            pltpu.SemaphoreType.DMA,
            pltpu.SemaphoreType.DMA,
        ],
    )
    return fn(src2d, dst2d, typ2d, x, zeros)


def _tc_body(x_ref, a_ref, wsl_ref, bsl_ref, w1_ref, b1_ref, g_ref, be_ref,
             w2_ref, b2_ref, o_ref):
    x = x_ref[...]
    acc = jnp.dot(x, wsl_ref[...],
                  preferred_element_type=jnp.float32) + bsl_ref[...][None, :]
    for r in range(_R):
        agg = jnp.concatenate(
            [a_ref[q, r * _QR:(r + 1) * _QR, :] for q in range(_NQ)], axis=0)
        h = x + agg
        h = jnp.dot(h, w1_ref[r],
                    preferred_element_type=jnp.float32) + b1_ref[r][None, :]
        mean = jnp.mean(h, axis=0)
        hc = h - mean[None, :]
        var = jnp.mean(hc * hc, axis=0)
        inv = lax.rsqrt(var + _BN_EPS)
        h = hc * (inv * g_ref[r])[None, :] + be_ref[r][None, :]
        h = jnp.maximum(h, 0.0)
        acc = acc + jnp.dot(h, w2_ref[r],
                            preferred_element_type=jnp.float32) + b2_ref[r][None, :]
    o_ref[...] = acc


def _tc_mlp(x, agg, W_sl, b_sl, W1, b1, gamma, beta, W2, b2):
    return pl.pallas_call(
        _tc_body,
        out_shape=jax.ShapeDtypeStruct((_N, _D), jnp.float32),
    )(x, agg, W_sl, b_sl, W1, b1, gamma, beta, W2, b2)


def kernel(x, edge_index, edge_type, W_sl, b_sl, W1, b1, gamma, beta, W2, b2):
    src = edge_index[0]
    dst = edge_index[1]
    pad = _E_PAD - _E
    src_p = jnp.concatenate([src, jnp.zeros((pad,), jnp.int32)])
    dst_p = jnp.concatenate([dst, jnp.full((pad,), _N, jnp.int32)])
    typ_p = jnp.concatenate([edge_type, jnp.zeros((pad,), jnp.int32)])
    src2d = src_p.reshape(_E_PAD // _CH, _CH)
    dst2d = dst_p.reshape(_E_PAD // _CH, _CH)
    typ2d = typ_p.reshape(_E_PAD // _CH, _CH)
    zeros = jnp.zeros((_RPT, _D), jnp.float32)
    agg = _run_sc_agg(src2d, dst2d, typ2d, x, zeros)
    return _tc_mlp(x, agg, W_sl, b_sl, W1, b1, gamma, beta, W2, b2)


# speedup vs baseline: 1.7103x; 1.0004x over previous
import jax
import jax.numpy as jnp
from jax import lax
from jax.experimental import pallas as pl
from jax.experimental.pallas import tpu as pltpu
from jax.experimental.pallas import tpu_sc as plsc

_N = 10000
_E = 320000
_D = 128
_R = 4
_BN_EPS = 1e-5

_NQ = 4
_QR = _N // _NQ
_ROWS = _R * _QR
_ROWS_PAD = 10240
_DUMMY = _ROWS
_NTILES = 16
_RPT = _ROWS_PAD // _NTILES
_CH = 128
_EPT = 20480
_E_PAD = _EPT * _NTILES
_NCH = _EPT // _CH
_GRP = 8
_NGRP = _NCH // _GRP


def _sc_agg_kernel(src_hbm, dst_hbm, typ_hbm, x_hbm, zero_hbm, out_hbm,
                   srcg0, srcg1, dstg0, dstg1, typg0, typg1,
                   slotg0, slotg1, rows0, rows1, agg_sh,
                   gsem0, gsem1, ssem0, ssem1):
    c = lax.axis_index("c")
    s = lax.axis_index("s")
    row_base = s * _NCH
    srcg = (srcg0, srcg1)
    dstg = (dstg0, dstg1)
    typg = (typg0, typg1)
    slotg = (slotg0, slotg1)
    rows = (rows0, rows1)
    gsem = (gsem0, gsem1)
    ssem = (ssem0, ssem1)

    def _fire_gather(u, t):
        pltpu.async_copy(x_hbm.at[srcg[u].at[t]], rows[t % 2], gsem[t % 2])

    def _wait_gather(u, t):
        pltpu.make_async_copy(x_hbm.at[srcg[u].at[t]], rows[t % 2],
                              gsem[t % 2]).wait()

    def _fire_scatter(u, t):
        pltpu.async_copy(rows[t % 2], agg_sh.at[slotg[u].at[t]],
                         ssem[t % 2], add=True)

    def _wait_scatter(u, t):
        pltpu.make_async_copy(rows[t % 2], agg_sh.at[slotg[u].at[t]],
                              ssem[t % 2]).wait()

    for p in range(2):
        q = c * 2 + p
        base_row = q * _QR

        zoff = pl.multiple_of(s * _RPT, _RPT)
        pltpu.sync_copy(zero_hbm, agg_sh.at[pl.ds(zoff, _RPT)])
        plsc.subcore_barrier()

        iota16 = lax.iota(jnp.int32, 16)

        def _slots(u, t):
            for k in range(_CH // 16):
                d = dstg[u][t, pl.ds(k * 16, 16)]
                tt = typg[u][t, pl.ds(k * 16, 16)]
                in_r = (d >= base_row) & (d < base_row + _QR)
                dummy = _DUMMY + (k % 15) * 16 + iota16
                slot = jnp.where(in_r, tt * _QR + (d - base_row), dummy)
                slotg[u][t, pl.ds(k * 16, 16)] = slot

        def _group(u, g):
            grow = row_base + g * _GRP
            pltpu.sync_copy(src_hbm.at[pl.ds(grow, _GRP)], srcg[u])
            pltpu.sync_copy(dst_hbm.at[pl.ds(grow, _GRP)], dstg[u])
            pltpu.sync_copy(typ_hbm.at[pl.ds(grow, _GRP)], typg[u])
            for t in range(_GRP):
                if t >= 2:
                    _wait_scatter(u, t - 2)
                else:
                    @pl.when(g > 0)
                    def _():
                        _wait_scatter(u ^ 1, _GRP - 2 + t)
                _slots(u, t)
                _fire_gather(u, t)
                if t >= 1:
                    _wait_gather(u, t - 1)
                    _fire_scatter(u, t - 1)
                else:
                    @pl.when(g > 0)
                    def _():
                        _wait_gather(u ^ 1, _GRP - 1)
                        _fire_scatter(u ^ 1, _GRP - 1)

        def _gpair(j, carry):
            _group(0, 2 * j)
            _group(1, 2 * j + 1)
            return carry
        lax.fori_loop(0, _NGRP // 2, _gpair, 0)
        _wait_gather(1, _GRP - 1)
        _fire_scatter(1, _GRP - 1)
        _wait_scatter(1, _GRP - 2)
        _wait_scatter(1, _GRP - 1)
        plsc.subcore_barrier()

        woff = pl.multiple_of(s * _RPT, _RPT)
        pltpu.sync_copy(agg_sh.at[pl.ds(woff, _RPT)],
                        out_hbm.at[q, pl.ds(woff, _RPT)])


def _run_sc_agg(src2d, dst2d, typ2d, x, zeros):
    mesh = plsc.VectorSubcoreMesh(core_axis_name="c", subcore_axis_name="s",
                                  num_cores=2)
    fn = pl.kernel(
        _sc_agg_kernel,
        mesh=mesh,
        out_type=jax.ShapeDtypeStruct((_NQ, _ROWS_PAD, _D), jnp.float32),
        scratch_types=[
            pltpu.VMEM((_GRP, _CH), jnp.int32),
            pltpu.VMEM((_GRP, _CH), jnp.int32),
            pltpu.VMEM((_GRP, _CH), jnp.int32),
            pltpu.VMEM((_GRP, _CH), jnp.int32),
            pltpu.VMEM((_GRP, _CH), jnp.int32),
            pltpu.VMEM((_GRP, _CH), jnp.int32),
            pltpu.VMEM((_GRP, _CH), jnp.int32),
            pltpu.VMEM((_GRP, _CH), jnp.int32),
            pltpu.VMEM((_CH, _D), jnp.float32),
            pltpu.VMEM((_CH, _D), jnp.float32),
            pltpu.VMEM_SHARED((_ROWS_PAD, _D), jnp.float32),
            pltpu.SemaphoreType.DMA,
            pltpu.SemaphoreType.DMA,
            pltpu.SemaphoreType.DMA,
            pltpu.SemaphoreType.DMA,
        ],
    )
    return fn(src2d, dst2d, typ2d, x, zeros)


def _tc_body(x_ref, a_ref, wsl_ref, bsl_ref, w1_ref, b1_ref, g_ref, be_ref,
             w2_ref, b2_ref, o_ref):
    x = x_ref[...]
    acc = jnp.dot(x, wsl_ref[...],
                  preferred_element_type=jnp.float32) + bsl_ref[...][None, :]
    for r in range(_R):
        agg = jnp.concatenate(
            [a_ref[q, r * _QR:(r + 1) * _QR, :] for q in range(_NQ)], axis=0)
        h = x + agg
        h = jnp.dot(h, w1_ref[r],
                    preferred_element_type=jnp.float32) + b1_ref[r][None, :]
        mean = jnp.mean(h, axis=0)
        hc = h - mean[None, :]
        var = jnp.mean(hc * hc, axis=0)
        inv = lax.rsqrt(var + _BN_EPS)
        h = hc * (inv * g_ref[r])[None, :] + be_ref[r][None, :]
        h = jnp.maximum(h, 0.0)
        acc = acc + jnp.dot(h, w2_ref[r],
                            preferred_element_type=jnp.float32) + b2_ref[r][None, :]
    o_ref[...] = acc


def _tc_mlp(x, agg, W_sl, b_sl, W1, b1, gamma, beta, W2, b2):
    return pl.pallas_call(
        _tc_body,
        out_shape=jax.ShapeDtypeStruct((_N, _D), jnp.float32),
    )(x, agg, W_sl, b_sl, W1, b1, gamma, beta, W2, b2)


def kernel(x, edge_index, edge_type, W_sl, b_sl, W1, b1, gamma, beta, W2, b2):
    src = edge_index[0]
    dst = edge_index[1]
    pad = _E_PAD - _E
    src_p = jnp.concatenate([src, jnp.zeros((pad,), jnp.int32)])
    dst_p = jnp.concatenate([dst, jnp.full((pad,), _N, jnp.int32)])
    typ_p = jnp.concatenate([edge_type, jnp.zeros((pad,), jnp.int32)])
    src2d = src_p.reshape(_E_PAD // _CH, _CH)
    dst2d = dst_p.reshape(_E_PAD // _CH, _CH)
    typ2d = typ_p.reshape(_E_PAD // _CH, _CH)
    zeros = jnp.zeros((_RPT, _D), jnp.float32)
    agg = _run_sc_agg(src2d, dst2d, typ2d, x, zeros)
    return _tc_mlp(x, agg, W_sl, b_sl, W1, b1, gamma, beta, W2, b2)
